# baseline (speedup 1.0000x reference)
import functools

import jax
import jax.numpy as jnp
from jax import lax
from jax.experimental import pallas as pl
from jax.experimental.pallas import tpu as pltpu
from jax.experimental.pallas import tpu_sc as plsc

_B = 4096
_V = 100000
_D = 64
_L = 16
_NC, _NS = 2, 16
_NW = _NC * _NS
_BPW = _B // _NW
_C = 16
_NCHUNK = _BPW // _C
_NSCORE = 21
_UROWS = _NSCORE * _C
_UIW = 112
_UIR = _UROWS // _UIW
_RB = 400


_NRB = _V // 2 // _RB


def _depad_body(vt_ref, vb_ref, ut_ref, ub_ref, vo_ref, uo_ref):
    vo_ref[:, 0:_D] = vt_ref[...]
    vo_ref[:, _D:2 * _D] = vb_ref[...]
    uo_ref[:, 0:_D] = ut_ref[...]
    uo_ref[:, _D:2 * _D] = ub_ref[...]


_depad = pl.pallas_call(
    _depad_body,
    grid=(_NRB,),
    in_specs=[
        pl.BlockSpec((_RB, _D), lambda i: (i, 0)),
        pl.BlockSpec((_RB, _D), lambda i: (i + _NRB, 0)),
        pl.BlockSpec((_RB, _D), lambda i: (i, 0)),
        pl.BlockSpec((_RB, _D), lambda i: (i + _NRB, 0)),
    ],
    out_specs=[pl.BlockSpec((_RB, 2 * _D), lambda i: (i, 0))] * 2,
    out_shape=[jax.ShapeDtypeStruct((_V // 2, 2 * _D), jnp.float32)] * 2,
)


def _sc_body(cpos_hbm, coff_hbm, upos_hbm, uoff_hbm, v_hbm, u_hbm, out_hbm,
             cpos_v, coff_v, upos_v, uoff_v, vrows, urows, out_v,
             sem0, sem1):
    sems = (sem0, sem1)
    wid = lax.axis_index("s") * _NC + lax.axis_index("c")

    def issue(g, b):
        base = wid * _BPW + g * _C
        pltpu.sync_copy(cpos_hbm.at[pl.ds(base * 4, _C * 4)], cpos_v.at[b])
        pltpu.sync_copy(coff_hbm.at[pl.ds(base * 4, _C * 4)],
                        coff_v.at[b, pl.ds(0, _C * 4)])
        pltpu.sync_copy(upos_hbm.at[pl.ds(base * _NSCORE, _UROWS)],
                        upos_v.at[b])
        pltpu.sync_copy(uoff_hbm.at[pl.ds(base * _NSCORE, _UROWS)],
                        uoff_v.at[b, pl.ds(0, _UROWS)])
        cps = [pltpu.async_copy(v_hbm.at[cpos_v.at[b]], vrows.at[b],
                                sems[b])]
        for i in range(_UIR):
            sl = pl.ds(i * _UIW, _UIW)
            cps.append(pltpu.async_copy(u_hbm.at[upos_v.at[b, sl]],
                                        urows.at[b, sl], sems[b]))
        return cps

    lanes = lax.iota(jnp.int32, _L)
    cps = issue(0, 0)
    for g in range(_NCHUNK):
        b = g % 2
        nxt = issue(g + 1, 1 - b) if g + 1 < _NCHUNK else []
        for cp in cps:
            cp.wait()
        cps = nxt

        def elem(c, carry, b=b):
            cvec = coff_v[b, pl.ds(4 * c, 16)]
            v = None
            for r in range(4):
                o = cvec[r]
                rv = [vrows[b, 4 * c + r, pl.ds(o + 16 * k, 16)]
                      for k in range(4)]
                v = rv if v is None else [v[k] + rv[k] for k in range(4)]
            v = [x * 0.25 for x in v]
            acc0 = jnp.zeros((_L,), jnp.float32)
            acc1 = jnp.zeros((_L,), jnp.float32)
            row = _NSCORE * c
            ovec0 = uoff_v[b, pl.ds(row, 16)]
            ovec1 = uoff_v[b, pl.ds(row + 16, 16)]
            for j in range(_NSCORE):
                o = ovec0[j] if j < 16 else ovec1[j - 16]
                p = urows[b, row + j, pl.ds(o, 16)] * v[0]
                for k in range(1, 4):
                    p = p + urows[b, row + j, pl.ds(o + 16 * k, 16)] * v[k]
                t = jnp.sum(p)
                t = t if j == 0 else -t
                if j < _L:
                    acc0 = jnp.where(lanes == j, t, acc0)
                else:
                    acc1 = jnp.where(lanes == (j - _L), t, acc1)
            out_v[c, pl.ds(0, _L)] = acc0
            out_v[c, pl.ds(_L, _L)] = acc1
            return carry

        lax.fori_loop(0, _C, elem, 0)
        base = wid * _BPW + g * _C
        pltpu.sync_copy(out_v, out_hbm.at[pl.ds(base, _C)])


_sc_call = functools.partial(
    pl.kernel,
    out_type=jax.ShapeDtypeStruct((_B, 2 * _L), jnp.float32),
    mesh=plsc.VectorSubcoreMesh(core_axis_name="c", subcore_axis_name="s"),
    scratch_types=[
        pltpu.VMEM((2, _C * 4), jnp.int32),
        pltpu.VMEM((2, _C * 4 + 16), jnp.int32),
        pltpu.VMEM((2, _UROWS), jnp.int32),
        pltpu.VMEM((2, _UROWS + 16), jnp.int32),
        pltpu.VMEM((2, _C * 4, 2 * _D), jnp.float32),
        pltpu.VMEM((2, _UROWS, 2 * _D), jnp.float32),
        pltpu.VMEM((_C, 2 * _L), jnp.float32),
        pltpu.SemaphoreType.DMA,
        pltpu.SemaphoreType.DMA,
    ],
    compiler_params=pltpu.CompilerParams(use_tc_tiling_on_sc=False,
                                         needs_layout_passes=False),
)(_sc_body)


def _tc_body(x_ref, o_ref):
    x = x_ref[...]
    col = lax.broadcasted_iota(jnp.int32, (_B, 2 * _L), 1)
    ls = jnp.minimum(x, 0.0) - jnp.log1p(jnp.exp(-jnp.abs(x)))
    ls = jnp.where(col < _NSCORE, ls, 0.0)
    o_ref[...] = jnp.full((1, 1), -jnp.sum(ls) / _B, jnp.float32)


_tc_call = pl.pallas_call(
    _tc_body,
    out_shape=jax.ShapeDtypeStruct((1, 1), jnp.float32),
)


def kernel(center_words, target_words, neg_words, V_w, U_w):
    cidx = center_words.astype(jnp.int32).reshape(-1)
    uidx = jnp.concatenate(
        [target_words.astype(jnp.int32), neg_words.astype(jnp.int32)],
        axis=1).reshape(-1)
    vw, uw = _depad(V_w, V_w, U_w, U_w)
    half = jnp.int32(_V // 2)
    cpos = jnp.where(cidx < half, cidx, cidx - half)
    coff = jnp.where(cidx < half, 0, _D).astype(jnp.int32)
    upos = jnp.where(uidx < half, uidx, uidx - half)
    uoff = jnp.where(uidx < half, 0, _D).astype(jnp.int32)
    scores = _sc_call(cpos, coff, upos, uoff, vw, uw)
    loss = _tc_call(scores)
    return loss[0, 0]


# speedup vs baseline: 1.5307x; 1.5307x over previous
import functools

import jax
import jax.numpy as jnp
from jax import lax
from jax.experimental import pallas as pl
from jax.experimental.pallas import tpu as pltpu
from jax.experimental.pallas import tpu_sc as plsc

_B = 4096
_V = 100000
_D = 64
_L = 16
_NC, _NS = 2, 16
_NW = _NC * _NS
_BPW = _B // _NW
_C = 64
_NCHUNK = _BPW // _C
_NSCORE = 21
_UROWS = _NSCORE * _C
_UIW = 112
_UIR = _UROWS // _UIW


def _sc_body(cidx_hbm, uidx_hbm, v_hbm, u_hbm, out_hbm,
             cidx_v, uidx_v, vrows, urows, out_v, sem0, sem1):
    sems = (sem0, sem1)
    wid = lax.axis_index("s") * _NC + lax.axis_index("c")

    def issue(g, b):
        base = wid * _BPW + g * _C
        pltpu.sync_copy(cidx_hbm.at[pl.ds(base * 4, _C * 4)], cidx_v.at[b])
        pltpu.sync_copy(uidx_hbm.at[pl.ds(base * _NSCORE, _UROWS)],
                        uidx_v.at[b])
        cps = [pltpu.async_copy(v_hbm.at[cidx_v.at[b]], vrows.at[b],
                                sems[b])]
        for i in range(_UIR):
            sl = pl.ds(i * _UIW, _UIW)
            cps.append(pltpu.async_copy(u_hbm.at[uidx_v.at[b, sl]],
                                        urows.at[b, sl], sems[b]))
        return cps

    lanes = lax.iota(jnp.int32, _L)
    cps = issue(0, 0)
    for g in range(_NCHUNK):
        b = g % 2
        nxt = issue(g + 1, 1 - b) if g + 1 < _NCHUNK else []
        for cp in cps:
            cp.wait()
        cps = nxt

        def elem(c, carry, b=b):
            sl = (pl.ds(0, 32), pl.ds(32, 32))

            def row_f32(ref, r):
                out = []
                for h in range(2):
                    out.extend(plsc.unpack(
                        ref[b, r, sl[h]],
                        format=plsc.PackFormat.INTERLEAVED))
                return out

            v = None
            for r in range(4):
                rv = row_f32(vrows, 4 * c + r)
                v = rv if v is None else [v[k] + rv[k] for k in range(4)]
            v = [x * 0.25 for x in v]
            acc0 = jnp.zeros((_L,), jnp.float32)
            acc1 = jnp.zeros((_L,), jnp.float32)
            row = _NSCORE * c
            for j in range(_NSCORE):
                u = row_f32(urows, row + j)
                p = u[0] * v[0]
                for k in range(1, 4):
                    p = p + u[k] * v[k]
                t = jnp.sum(p)
                t = t if j == 0 else -t
                if j < _L:
                    acc0 = jnp.where(lanes == j, t, acc0)
                else:
                    acc1 = jnp.where(lanes == (j - _L), t, acc1)
            out_v[c, pl.ds(0, _L)] = acc0
            out_v[c, pl.ds(_L, _L)] = acc1
            return carry

        lax.fori_loop(0, _C, elem, 0)
        base = wid * _BPW + g * _C
        pltpu.sync_copy(out_v, out_hbm.at[pl.ds(base, _C)])


_sc_call = functools.partial(
    pl.kernel,
    out_type=jax.ShapeDtypeStruct((_B, 2 * _L), jnp.float32),
    mesh=plsc.VectorSubcoreMesh(core_axis_name="c", subcore_axis_name="s"),
    scratch_types=[
        pltpu.VMEM((2, _C * 4), jnp.int32),
        pltpu.VMEM((2, _UROWS), jnp.int32),
        pltpu.VMEM((2, _C * 4, _D), jnp.bfloat16),
        pltpu.VMEM((2, _UROWS, _D), jnp.bfloat16),
        pltpu.VMEM((_C, 2 * _L), jnp.float32),
        pltpu.SemaphoreType.DMA,
        pltpu.SemaphoreType.DMA,
    ],
    compiler_params=pltpu.CompilerParams(use_tc_tiling_on_sc=False,
                                         needs_layout_passes=False),
)(_sc_body)


def _tc_body(x_ref, o_ref):
    x = x_ref[...]
    col = lax.broadcasted_iota(jnp.int32, x.shape, 1)
    ls = jnp.minimum(x, 0.0) - jnp.log1p(jnp.exp(-jnp.abs(x)))
    ls = jnp.where((col & (2 * _L - 1)) < _NSCORE, ls, 0.0)
    o_ref[...] = jnp.full((1, 1), -jnp.sum(ls) / _B, jnp.float32)


_tc_call = pl.pallas_call(
    _tc_body,
    out_shape=jax.ShapeDtypeStruct((1, 1), jnp.float32),
)


def kernel(center_words, target_words, neg_words, V_w, U_w):
    cidx = center_words.astype(jnp.int32).reshape(-1)
    uidx = jnp.concatenate(
        [target_words.astype(jnp.int32), neg_words.astype(jnp.int32)],
        axis=1).reshape(-1)
    scores = _sc_call(cidx, uidx, V_w.astype(jnp.bfloat16),
                      U_w.astype(jnp.bfloat16))
    loss = _tc_call(scores.reshape(_B // 4, 8 * _L))
    return loss[0, 0]


# speedup vs baseline: 1.9324x; 1.2624x over previous
import functools

import jax
import jax.numpy as jnp
from jax import lax
from jax.experimental import pallas as pl
from jax.experimental.pallas import tpu as pltpu
from jax.experimental.pallas import tpu_sc as plsc

_B = 4096
_V = 100000
_D = 64
_L = 16
_NC, _NS = 2, 16
_NW = _NC * _NS
_BPW = _B // _NW
_C = 32
_NCHUNK = _BPW // _C
_NSCORE = 21
_UROWS = _NSCORE * _C
_UIW = 96
_UIR = _UROWS // _UIW


def _sc_body(cidx_hbm, uidx_hbm, v_hbm, u_hbm, out_hbm,
             cidx_v, uidx_v, vrows, urows, out_v, sem0, sem1):
    sems = (sem0, sem1)
    wid = lax.axis_index("s") * _NC + lax.axis_index("c")

    def issue(g, b):
        base = wid * _BPW + g * _C
        pltpu.sync_copy(cidx_hbm.at[pl.ds(base * 4, _C * 4)], cidx_v.at[b])
        pltpu.sync_copy(uidx_hbm.at[pl.ds(base * _NSCORE, _UROWS)],
                        uidx_v.at[b])
        cps = [pltpu.async_copy(v_hbm.at[cidx_v.at[b]], vrows.at[b],
                                sems[b])]
        for i in range(_UIR):
            sl = pl.ds(i * _UIW, _UIW)
            cps.append(pltpu.async_copy(u_hbm.at[uidx_v.at[b, sl]],
                                        urows.at[b, sl], sems[b]))
        return cps

    lanes = lax.iota(jnp.int32, _L)
    cps = issue(0, 0)
    for g in range(_NCHUNK):
        b = g % 2
        nxt = issue(g + 1, 1 - b) if g + 1 < _NCHUNK else []
        for cp in cps:
            cp.wait()
        cps = nxt

        def elem(c, carry, b=b):
            sl = [pl.ds(16 * k, 16) for k in range(4)]
            v = [(vrows[b, 4 * c, s] + vrows[b, 4 * c + 1, s]
                  + vrows[b, 4 * c + 2, s] + vrows[b, 4 * c + 3, s]) * 0.25
                 for s in sl]
            acc0 = jnp.zeros((_L,), jnp.float32)
            acc1 = jnp.zeros((_L,), jnp.float32)
            row = _NSCORE * c
            for j in range(_NSCORE):
                p = urows[b, row + j, sl[0]] * v[0]
                for k in range(1, 4):
                    p = p + urows[b, row + j, sl[k]] * v[k]
                t = jnp.sum(p)
                t = t if j == 0 else -t
                if j < _L:
                    acc0 = jnp.where(lanes == j, t, acc0)
                else:
                    acc1 = jnp.where(lanes == (j - _L), t, acc1)
            out_v[c, pl.ds(0, _L)] = acc0
            out_v[c, pl.ds(_L, _L)] = acc1
            return carry

        lax.fori_loop(0, _C, elem, 0)
        base = wid * _BPW + g * _C
        pltpu.sync_copy(out_v, out_hbm.at[pl.ds(base, _C)])


_sc_call = functools.partial(
    pl.kernel,
    out_type=jax.ShapeDtypeStruct((_B, 2 * _L), jnp.float32),
    mesh=plsc.VectorSubcoreMesh(core_axis_name="c", subcore_axis_name="s"),
    scratch_types=[
        pltpu.VMEM((2, _C * 4), jnp.int32),
        pltpu.VMEM((2, _UROWS), jnp.int32),
        pltpu.VMEM((2, _C * 4, _D), jnp.float32),
        pltpu.VMEM((2, _UROWS, _D), jnp.float32),
        pltpu.VMEM((_C, 2 * _L), jnp.float32),
        pltpu.SemaphoreType.DMA,
        pltpu.SemaphoreType.DMA,
    ],
    compiler_params=pltpu.CompilerParams(use_tc_tiling_on_sc=False,
                                         needs_layout_passes=False),
)(_sc_body)


def _tc_body(x_ref, o_ref):
    x = x_ref[...]
    col = lax.broadcasted_iota(jnp.int32, x.shape, 1)
    ls = jnp.minimum(x, 0.0) - jnp.log1p(jnp.exp(-jnp.abs(x)))
    ls = jnp.where((col & (2 * _L - 1)) < _NSCORE, ls, 0.0)
    o_ref[...] = jnp.full((1, 1), -jnp.sum(ls) / _B, jnp.float32)


_tc_call = pl.pallas_call(
    _tc_body,
    out_shape=jax.ShapeDtypeStruct((1, 1), jnp.float32),
)


def kernel(center_words, target_words, neg_words, V_w, U_w):
    cidx = center_words.astype(jnp.int32).reshape(-1)
    uidx = jnp.concatenate(
        [target_words.astype(jnp.int32), neg_words.astype(jnp.int32)],
        axis=1).reshape(-1)
    scores = _sc_call(cidx, uidx, V_w, U_w)
    loss = _tc_call(scores.reshape(_B // 4, 8 * _L))
    return loss[0, 0]
